# Initial kernel scaffold; baseline (speedup 1.0000x reference)
import jax
import jax.numpy as jnp
from jax.experimental import pallas as pl


def kernel(x, edge_index, edge_type, W1, W_root1, b1, W2, W_root2, b2):
    raise NotImplementedError("write your pallas kernel here")


# speedup vs baseline: 10.0167x; 10.0167x over previous
import functools

import jax
import jax.numpy as jnp
from jax import lax
from jax.experimental import pallas as pl
from jax.experimental.pallas import tpu as pltpu
from jax.experimental.pallas import tpu_sc as plsc

N = 10000
E = 320000
R = 8
C = 128

NC = 2
NS = 16
NW = NC * NS

B = 80
EDGES_PER_TILE = E // NS
EDGES_PER_WORKER = E // NW
CNT_PAD = 81920
N_ACC = 10240
ROWS_PER_TILE = N_ACC // NS
ZROWS = 128

_mesh = plsc.VectorSubcoreMesh(core_axis_name="c", subcore_axis_name="s")


def _fill_const(ref, n16, value, dtype):
    v = jnp.full((16,), value, dtype=dtype)

    def body(i, _):
        ref[pl.ds(i * 16, 16)] = v
        return 0

    lax.fori_loop(0, n16, body, 0)


@functools.partial(
    pl.kernel,
    out_type=[
        jax.ShapeDtypeStruct((E,), jnp.float32),
        jax.ShapeDtypeStruct((E,), jnp.int32),
    ],
    mesh=_mesh,
    scratch_types=[
        pltpu.VMEM_SHARED((CNT_PAD,), jnp.float32),
        pltpu.VMEM((CNT_PAD // NS,), jnp.float32),
        pltpu.VMEM((B,), jnp.int32),
        pltpu.VMEM((B,), jnp.int32),
        pltpu.VMEM((B,), jnp.int32),
        pltpu.VMEM((B,), jnp.int32),
        pltpu.VMEM((B,), jnp.float32),
        pltpu.VMEM((B,), jnp.float32),
        pltpu.VMEM((B,), jnp.int32),
    ],
)
def _sc_structure(src_hbm, dst_hbm, et_hbm, norm_hbm, rid_hbm,
                  cnt_sh, zbuf, d_v, t_v, s_v, comb_v, f_v, nrm_v, rid_v):
    cid = lax.axis_index("c")
    sid = lax.axis_index("s")
    wid = sid * NC + cid

    n16 = (CNT_PAD // NS) // 16
    _fill_const(zbuf, n16, 0.0, jnp.float32)
    pltpu.sync_copy(zbuf, cnt_sh.at[pl.ds(sid * (CNT_PAD // NS), CNT_PAD // NS)])
    plsc.subcore_barrier()

    _fill_const(f_v, B // 16, 1.0, jnp.float32)
    base = sid * EDGES_PER_TILE

    def count_block(b, _):
        off = base + b * B
        pltpu.sync_copy(dst_hbm.at[pl.ds(off, B)], d_v)
        pltpu.sync_copy(et_hbm.at[pl.ds(off, B)], t_v)
        for k in range(B // 16):
            sl = pl.ds(k * 16, 16)
            comb_v[sl] = d_v[sl] * R + t_v[sl]
        pltpu.sync_copy(f_v, cnt_sh.at[comb_v], add=True)
        return 0

    lax.fori_loop(0, EDGES_PER_TILE // B, count_block, 0)
    plsc.subcore_barrier()

    base2 = wid * EDGES_PER_WORKER

    def norm_block(b, _):
        off = base2 + b * B
        pltpu.sync_copy(dst_hbm.at[pl.ds(off, B)], d_v)
        pltpu.sync_copy(et_hbm.at[pl.ds(off, B)], t_v)
        pltpu.sync_copy(src_hbm.at[pl.ds(off, B)], s_v)
        for k in range(B // 16):
            sl = pl.ds(k * 16, 16)
            comb_v[sl] = d_v[sl] * R + t_v[sl]
            rid_v[sl] = t_v[sl] * N + s_v[sl]
        pltpu.sync_copy(rid_v, rid_hbm.at[pl.ds(off, B)])
        pltpu.sync_copy(cnt_sh.at[comb_v], f_v)
        for k in range(B // 16):
            sl = pl.ds(k * 16, 16)
            nrm_v[sl] = 1.0 / jnp.maximum(f_v[sl], 1.0)
        pltpu.sync_copy(nrm_v, norm_hbm.at[pl.ds(off, B)])
        return 0

    lax.fori_loop(0, EDGES_PER_WORKER // B, norm_block, 0)


@functools.partial(
    pl.kernel,
    out_type=jax.ShapeDtypeStruct((NC, N_ACC, C), jnp.float32),
    mesh=_mesh,
    scratch_types=[
        pltpu.VMEM_SHARED((N_ACC, C), jnp.float32),
        pltpu.VMEM((ZROWS, C), jnp.float32),
        pltpu.VMEM((B,), jnp.int32),
        pltpu.VMEM((B,), jnp.int32),
        pltpu.VMEM((B,), jnp.float32),
        pltpu.VMEM((B, C), jnp.float32),
    ],
)
def _sc_aggregate(hall_hbm, rid_hbm, dst_hbm, nrm_hbm, out_hbm,
                  acc_sh, zbuf, rid_v, dst_v, nrm_v, rows_v):
    cid = lax.axis_index("c")
    sid = lax.axis_index("s")
    wid = sid * NC + cid

    def zrow(i, _):
        for k in range(C // 16):
            zbuf[i, pl.ds(k * 16, 16)] = jnp.zeros((16,), jnp.float32)
        return 0

    lax.fori_loop(0, ZROWS, zrow, 0)
    for i in range(ROWS_PER_TILE // ZROWS):
        pltpu.sync_copy(zbuf, acc_sh.at[pl.ds(sid * ROWS_PER_TILE + i * ZROWS, ZROWS)])
    plsc.subcore_barrier()

    base = wid * EDGES_PER_WORKER

    def edge_block(b, _):
        off = base + b * B
        pltpu.sync_copy(rid_hbm.at[pl.ds(off, B)], rid_v)
        pltpu.sync_copy(dst_hbm.at[pl.ds(off, B)], dst_v)
        pltpu.sync_copy(nrm_hbm.at[pl.ds(off, B)], nrm_v)
        pltpu.sync_copy(hall_hbm.at[rid_v], rows_v)

        def scale_chunk(g, _):
            nv = nrm_v[pl.ds(g * 16, 16)]
            for j in range(16):
                row = g * 16 + j
                s = nv[j]
                for k in range(C // 16):
                    sl = pl.ds(k * 16, 16)
                    rows_v[row, sl] = rows_v[row, sl] * s
            return 0

        lax.fori_loop(0, B // 16, scale_chunk, 0)
        pltpu.sync_copy(rows_v, acc_sh.at[dst_v], add=True)
        return 0

    lax.fori_loop(0, EDGES_PER_WORKER // B, edge_block, 0)
    plsc.subcore_barrier()

    for i in range(ROWS_PER_TILE // ZROWS):
        r0 = sid * ROWS_PER_TILE + i * ZROWS
        pltpu.sync_copy(acc_sh.at[pl.ds(r0, ZROWS)], out_hbm.at[cid, pl.ds(r0, ZROWS)])


BN = 1000


def _rel_matmul_body(x_ref, w_ref, o_ref):
    o_ref[0] = jnp.dot(x_ref[...], w_ref[0], preferred_element_type=jnp.float32)


def _rel_matmul(x, W):
    return pl.pallas_call(
        _rel_matmul_body,
        grid=(R, N // BN),
        in_specs=[
            pl.BlockSpec((BN, C), lambda r, n: (n, 0)),
            pl.BlockSpec((1, C, C), lambda r, n: (r, 0, 0)),
        ],
        out_specs=pl.BlockSpec((1, BN, C), lambda r, n: (r, n, 0)),
        out_shape=jax.ShapeDtypeStruct((R, N, C), jnp.float32),
    )(x, W)


def _combine_relu_body(p0_ref, p1_ref, x_ref, w_ref, b_ref, o_ref):
    h = p0_ref[0] + p1_ref[0]
    h = h + jnp.dot(x_ref[...], w_ref[...], preferred_element_type=jnp.float32)
    h = h + b_ref[...]
    o_ref[...] = jnp.maximum(h, 0.0)


def _combine_norm_body(p0_ref, p1_ref, x_ref, w_ref, b_ref, o_ref):
    h = p0_ref[0] + p1_ref[0]
    h = h + jnp.dot(x_ref[...], w_ref[...], preferred_element_type=jnp.float32)
    h = h + b_ref[...]
    nrm = jnp.sqrt(jnp.sum(h * h, axis=-1, keepdims=True))
    o_ref[...] = h / jnp.maximum(nrm, 1e-12)


def _combine_call(parts, x, w_root, b, body):
    return pl.pallas_call(
        body,
        grid=(N // BN,),
        in_specs=[
            pl.BlockSpec((1, BN, C), lambda n: (0, n, 0)),
            pl.BlockSpec((1, BN, C), lambda n: (0, n, 0)),
            pl.BlockSpec((BN, C), lambda n: (n, 0)),
            pl.BlockSpec((C, C), lambda n: (0, 0)),
            pl.BlockSpec((1, C), lambda n: (0, 0)),
        ],
        out_specs=pl.BlockSpec((BN, C), lambda n: (n, 0)),
        out_shape=jax.ShapeDtypeStruct((N, C), jnp.float32),
    )(parts[0:1], parts[1:2], x, w_root, b[None, :])


def _layer(x, W, w_root, b, rid, dst, nrm, body):
    h_all = _rel_matmul(x, W).reshape(R * N, C)
    parts = _sc_aggregate(h_all, rid, dst, nrm)
    return _combine_call(parts, x, w_root, b, body)


@jax.jit
def kernel(x, edge_index, edge_type, W1, W_root1, b1, W2, W_root2, b2):
    src = edge_index[0]
    dst = edge_index[1]
    et = edge_type
    nrm, rid = _sc_structure(src, dst, et)
    h = _layer(x, W1, W_root1, b1, rid, dst, nrm, _combine_relu_body)
    out = _layer(h, W2, W_root2, b2, rid, dst, nrm, _combine_norm_body)
    return out


# speedup vs baseline: 16.2669x; 1.6240x over previous
import functools

import jax
import jax.numpy as jnp
from jax import lax
from jax.experimental import pallas as pl
from jax.experimental.pallas import tpu as pltpu
from jax.experimental.pallas import tpu_sc as plsc

N = 10000
E = 320000
R = 8
C = 128

NC = 2
NS = 16
NW = NC * NS

B = 80
EDGES_PER_TILE = E // NS
EDGES_PER_WORKER = E // NW
CNT_PAD = 81920
N_ACC = 10240
ROWS_PER_TILE = N_ACC // NS
ZROWS = 128

_mesh = plsc.VectorSubcoreMesh(core_axis_name="c", subcore_axis_name="s")


def _fill_const(ref, n16, value, dtype):
    v = jnp.full((16,), value, dtype=dtype)

    def body(i, _):
        ref[pl.ds(i * 16, 16)] = v
        return 0

    lax.fori_loop(0, n16, body, 0)


@functools.partial(
    pl.kernel,
    out_type=[
        jax.ShapeDtypeStruct((E,), jnp.float32),
        jax.ShapeDtypeStruct((E,), jnp.int32),
    ],
    mesh=_mesh,
    scratch_types=[
        pltpu.VMEM_SHARED((CNT_PAD,), jnp.float32),
        pltpu.VMEM((CNT_PAD // NS,), jnp.float32),
        pltpu.VMEM((B,), jnp.int32),
        pltpu.VMEM((B,), jnp.int32),
        pltpu.VMEM((B,), jnp.int32),
        pltpu.VMEM((B,), jnp.int32),
        pltpu.VMEM((B,), jnp.float32),
        pltpu.VMEM((B,), jnp.float32),
        pltpu.VMEM((B,), jnp.int32),
    ],
)
def _sc_structure(src_hbm, dst_hbm, et_hbm, norm_hbm, rid_hbm,
                  cnt_sh, zbuf, d_v, t_v, s_v, comb_v, f_v, nrm_v, rid_v):
    cid = lax.axis_index("c")
    sid = lax.axis_index("s")
    wid = sid * NC + cid

    n16 = (CNT_PAD // NS) // 16
    _fill_const(zbuf, n16, 0.0, jnp.float32)
    pltpu.sync_copy(zbuf, cnt_sh.at[pl.ds(sid * (CNT_PAD // NS), CNT_PAD // NS)])
    plsc.subcore_barrier()

    _fill_const(f_v, B // 16, 1.0, jnp.float32)
    base = sid * EDGES_PER_TILE

    def count_block(b, _):
        off = base + b * B
        pltpu.sync_copy(dst_hbm.at[pl.ds(off, B)], d_v)
        pltpu.sync_copy(et_hbm.at[pl.ds(off, B)], t_v)
        for k in range(B // 16):
            sl = pl.ds(k * 16, 16)
            comb_v[sl] = d_v[sl] * R + t_v[sl]
        pltpu.sync_copy(f_v, cnt_sh.at[comb_v], add=True)
        return 0

    lax.fori_loop(0, EDGES_PER_TILE // B, count_block, 0)
    plsc.subcore_barrier()

    base2 = wid * EDGES_PER_WORKER

    def norm_block(b, _):
        off = base2 + b * B
        pltpu.sync_copy(dst_hbm.at[pl.ds(off, B)], d_v)
        pltpu.sync_copy(et_hbm.at[pl.ds(off, B)], t_v)
        pltpu.sync_copy(src_hbm.at[pl.ds(off, B)], s_v)
        for k in range(B // 16):
            sl = pl.ds(k * 16, 16)
            comb_v[sl] = d_v[sl] * R + t_v[sl]
            rid_v[sl] = t_v[sl] * N + s_v[sl]
        pltpu.sync_copy(rid_v, rid_hbm.at[pl.ds(off, B)])
        pltpu.sync_copy(cnt_sh.at[comb_v], f_v)
        for k in range(B // 16):
            sl = pl.ds(k * 16, 16)
            nrm_v[sl] = 1.0 / jnp.maximum(f_v[sl], 1.0)
        pltpu.sync_copy(nrm_v, norm_hbm.at[pl.ds(off, B)])
        return 0

    lax.fori_loop(0, EDGES_PER_WORKER // B, norm_block, 0)


NB = EDGES_PER_WORKER // B


@functools.partial(
    pl.kernel,
    out_type=jax.ShapeDtypeStruct((NC, N_ACC, C), jnp.float32),
    mesh=_mesh,
    scratch_types=[
        pltpu.VMEM_SHARED((N_ACC, C), jnp.float32),
        pltpu.VMEM((EDGES_PER_WORKER,), jnp.int32),
        pltpu.VMEM((EDGES_PER_WORKER,), jnp.float32),
        pltpu.VMEM((B,), jnp.int32),
        pltpu.VMEM((B,), jnp.int32),
        pltpu.VMEM((B,), jnp.int32),
        pltpu.VMEM((B,), jnp.int32),
        pltpu.VMEM((B, C), jnp.float32),
        pltpu.VMEM((B, C), jnp.float32),
        pltpu.SemaphoreType.DMA,
        pltpu.SemaphoreType.DMA,
        pltpu.SemaphoreType.DMA,
        pltpu.SemaphoreType.DMA,
    ],
)
def _sc_aggregate(hall_hbm, rid_hbm, dst_hbm, nrm_hbm, out_hbm,
                  acc_sh, rid_all, nrm_all,
                  rid_v0, rid_v1, dst_v0, dst_v1, rows0, rows1,
                  semg0, semg1, semd0, semd1):
    cid = lax.axis_index("c")
    sid = lax.axis_index("s")
    wid = sid * NC + cid
    base = wid * EDGES_PER_WORKER

    def zrow(i, _):
        for k in range(C // 16):
            rows0[i, pl.ds(k * 16, 16)] = jnp.zeros((16,), jnp.float32)
        return 0

    lax.fori_loop(0, B, zrow, 0)
    for i in range(ROWS_PER_TILE // B):
        pltpu.sync_copy(rows0, acc_sh.at[pl.ds(sid * ROWS_PER_TILE + i * B, B)])
    plsc.subcore_barrier()

    pltpu.sync_copy(rid_hbm.at[pl.ds(base, EDGES_PER_WORKER)], rid_all)
    pltpu.sync_copy(nrm_hbm.at[pl.ds(base, EDGES_PER_WORKER)], nrm_all)

    def fire(b, rid_buf, rows_buf, dst_buf, semg, semd):
        def cp(k, _):
            rid_buf[pl.ds(k * 16, 16)] = rid_all[pl.ds(b * B + k * 16, 16)]
            return 0

        lax.fori_loop(0, B // 16, cp, 0)
        pltpu.async_copy(hall_hbm.at[rid_buf], rows_buf, semg)
        pltpu.async_copy(dst_hbm.at[pl.ds(base + b * B, B)], dst_buf, semd)

    def consume(b, rid_buf, rows_buf, dst_buf, semg, semd):
        pltpu.make_async_copy(hall_hbm.at[rid_buf], rows_buf, semg).wait()

        def scale_chunk(g, _):
            nv = nrm_all[pl.ds(b * B + g * 16, 16)]
            for j in range(16):
                row = g * 16 + j
                s = nv[j]
                for k in range(C // 16):
                    sl = pl.ds(k * 16, 16)
                    rows_buf[row, sl] = rows_buf[row, sl] * s
            return 0

        lax.fori_loop(0, B // 16, scale_chunk, 0)
        pltpu.make_async_copy(
            dst_hbm.at[pl.ds(base + b * B, B)], dst_buf, semd).wait()
        pltpu.sync_copy(rows_buf, acc_sh.at[dst_buf], add=True)

    fire(0, rid_v0, rows0, dst_v0, semg0, semd0)

    def pipe(g, _):
        b0 = g * 2
        fire(b0 + 1, rid_v1, rows1, dst_v1, semg1, semd1)
        consume(b0, rid_v0, rows0, dst_v0, semg0, semd0)
        fire(b0 + 2, rid_v0, rows0, dst_v0, semg0, semd0)
        consume(b0 + 1, rid_v1, rows1, dst_v1, semg1, semd1)
        return 0

    lax.fori_loop(0, (NB - 1) // 2, pipe, 0)
    consume(NB - 1, rid_v0, rows0, dst_v0, semg0, semd0)
    plsc.subcore_barrier()

    for i in range(ROWS_PER_TILE // ZROWS):
        r0 = sid * ROWS_PER_TILE + i * ZROWS
        pltpu.sync_copy(acc_sh.at[pl.ds(r0, ZROWS)], out_hbm.at[cid, pl.ds(r0, ZROWS)])


BN = 1000


def _rel_matmul_body(x_ref, w_ref, o_ref):
    o_ref[0] = jnp.dot(x_ref[...], w_ref[0], preferred_element_type=jnp.float32)


def _rel_matmul(x, W):
    return pl.pallas_call(
        _rel_matmul_body,
        grid=(R, N // BN),
        in_specs=[
            pl.BlockSpec((BN, C), lambda r, n: (n, 0)),
            pl.BlockSpec((1, C, C), lambda r, n: (r, 0, 0)),
        ],
        out_specs=pl.BlockSpec((1, BN, C), lambda r, n: (r, n, 0)),
        out_shape=jax.ShapeDtypeStruct((R, N, C), jnp.float32),
    )(x, W)


def _combine_relu_body(p0_ref, p1_ref, x_ref, w_ref, b_ref, o_ref):
    h = p0_ref[0] + p1_ref[0]
    h = h + jnp.dot(x_ref[...], w_ref[...], preferred_element_type=jnp.float32)
    h = h + b_ref[...]
    o_ref[...] = jnp.maximum(h, 0.0)


def _combine_norm_body(p0_ref, p1_ref, x_ref, w_ref, b_ref, o_ref):
    h = p0_ref[0] + p1_ref[0]
    h = h + jnp.dot(x_ref[...], w_ref[...], preferred_element_type=jnp.float32)
    h = h + b_ref[...]
    nrm = jnp.sqrt(jnp.sum(h * h, axis=-1, keepdims=True))
    o_ref[...] = h / jnp.maximum(nrm, 1e-12)


def _combine_call(parts, x, w_root, b, body):
    return pl.pallas_call(
        body,
        grid=(N // BN,),
        in_specs=[
            pl.BlockSpec((1, BN, C), lambda n: (0, n, 0)),
            pl.BlockSpec((1, BN, C), lambda n: (0, n, 0)),
            pl.BlockSpec((BN, C), lambda n: (n, 0)),
            pl.BlockSpec((C, C), lambda n: (0, 0)),
            pl.BlockSpec((1, C), lambda n: (0, 0)),
        ],
        out_specs=pl.BlockSpec((BN, C), lambda n: (n, 0)),
        out_shape=jax.ShapeDtypeStruct((N, C), jnp.float32),
    )(parts[0:1], parts[1:2], x, w_root, b[None, :])


def _layer(x, W, w_root, b, rid, dst, nrm, body):
    h_all = _rel_matmul(x, W).reshape(R * N, C)
    parts = _sc_aggregate(h_all, rid, dst, nrm)
    return _combine_call(parts, x, w_root, b, body)


@jax.jit
def kernel(x, edge_index, edge_type, W1, W_root1, b1, W2, W_root2, b2):
    src = edge_index[0]
    dst = edge_index[1]
    et = edge_type
    nrm, rid = _sc_structure(src, dst, et)
    h = _layer(x, W1, W_root1, b1, rid, dst, nrm, _combine_relu_body)
    out = _layer(h, W2, W_root2, b2, rid, dst, nrm, _combine_norm_body)
    return out


# speedup vs baseline: 28.9459x; 1.7794x over previous
import functools

import jax
import jax.numpy as jnp
from jax import lax
from jax.experimental import pallas as pl
from jax.experimental.pallas import tpu as pltpu
from jax.experimental.pallas import tpu_sc as plsc

N = 10000
E = 320000
R = 8
C = 128

NC = 2
NS = 16
NW = NC * NS

B = 80
EDGES_PER_TILE = E // NS
EDGES_PER_WORKER = E // NW
CNT_PAD = 81920
N_ACC = 10240
ROWS_PER_TILE = N_ACC // NS
ZROWS = 128

_mesh = plsc.VectorSubcoreMesh(core_axis_name="c", subcore_axis_name="s")


def _fill_const(ref, n16, value, dtype):
    v = jnp.full((16,), value, dtype=dtype)

    def body(i, _):
        ref[pl.ds(i * 16, 16)] = v
        return 0

    lax.fori_loop(0, n16, body, 0)


NBT = EDGES_PER_TILE // B
NBW = EDGES_PER_WORKER // B
W_IND = 5


@functools.partial(
    pl.kernel,
    out_type=[
        jax.ShapeDtypeStruct((E,), jnp.float32),
        jax.ShapeDtypeStruct((E,), jnp.int32),
    ],
    mesh=_mesh,
    scratch_types=[
        pltpu.VMEM_SHARED((CNT_PAD,), jnp.float32),
        pltpu.VMEM((CNT_PAD // NS,), jnp.float32),
        pltpu.VMEM((EDGES_PER_TILE,), jnp.int32),
        pltpu.VMEM((EDGES_PER_TILE,), jnp.int32),
        pltpu.VMEM((EDGES_PER_WORKER,), jnp.int32),
        pltpu.VMEM((NBT, B), jnp.int32),
        pltpu.VMEM((B,), jnp.float32),
        pltpu.VMEM((NBW, B), jnp.float32),
        pltpu.VMEM((EDGES_PER_WORKER,), jnp.float32),
        pltpu.VMEM((EDGES_PER_WORKER,), jnp.int32),
        pltpu.SemaphoreType.DMA,
    ],
)
def _sc_structure(src_hbm, dst_hbm, et_hbm, norm_hbm, rid_hbm,
                  cnt_sh, zbuf, d_all, t_all, s_all, comb2, ones_v,
                  cval2, nrm_st, rid_st, sem):
    cid = lax.axis_index("c")
    sid = lax.axis_index("s")
    wid = sid * NC + cid

    n16 = (CNT_PAD // NS) // 16
    _fill_const(zbuf, n16, 0.0, jnp.float32)
    pltpu.sync_copy(zbuf, cnt_sh.at[pl.ds(sid * (CNT_PAD // NS), CNT_PAD // NS)])
    _fill_const(ones_v, B // 16, 1.0, jnp.float32)
    plsc.subcore_barrier()

    base = sid * EDGES_PER_TILE
    pltpu.sync_copy(dst_hbm.at[pl.ds(base, EDGES_PER_TILE)], d_all)
    pltpu.sync_copy(et_hbm.at[pl.ds(base, EDGES_PER_TILE)], t_all)

    def mkcomb(b, _):
        for k in range(B // 16):
            sl = pl.ds(b * B + k * 16, 16)
            comb2[b, pl.ds(k * 16, 16)] = d_all[sl] * R + t_all[sl]
        return 0

    lax.fori_loop(0, NBT, mkcomb, 0)

    def cnt_wave(g, _):
        for j in range(W_IND):
            pltpu.async_copy(ones_v, cnt_sh.at[comb2.at[g * W_IND + j]], sem,
                             add=True)
        for j in range(W_IND):
            pltpu.make_async_copy(ones_v, cnt_sh.at[comb2.at[g * W_IND + j]],
                                  sem).wait()
        return 0

    lax.fori_loop(0, NBT // W_IND, cnt_wave, 0)
    plsc.subcore_barrier()

    base2 = wid * EDGES_PER_WORKER
    pltpu.sync_copy(dst_hbm.at[pl.ds(base2, EDGES_PER_WORKER)],
                    d_all.at[pl.ds(0, EDGES_PER_WORKER)])
    pltpu.sync_copy(et_hbm.at[pl.ds(base2, EDGES_PER_WORKER)],
                    t_all.at[pl.ds(0, EDGES_PER_WORKER)])
    pltpu.sync_copy(src_hbm.at[pl.ds(base2, EDGES_PER_WORKER)], s_all)

    def mkidx(b, _):
        for k in range(B // 16):
            sl = pl.ds(b * B + k * 16, 16)
            t = t_all[sl]
            comb2[b, pl.ds(k * 16, 16)] = d_all[sl] * R + t
            rid_st[sl] = t * N + s_all[sl]
        return 0

    lax.fori_loop(0, NBW, mkidx, 0)
    pltpu.sync_copy(rid_st, rid_hbm.at[pl.ds(base2, EDGES_PER_WORKER)])

    def gat_wave(g, _):
        for j in range(W_IND):
            b = g * W_IND + j
            pltpu.async_copy(cnt_sh.at[comb2.at[b]], cval2.at[b], sem)
        for j in range(W_IND):
            b = g * W_IND + j
            pltpu.make_async_copy(cnt_sh.at[comb2.at[b]], cval2.at[b],
                                  sem).wait()
        return 0

    lax.fori_loop(0, NBW // W_IND, gat_wave, 0)

    def mknrm(b, _):
        for k in range(B // 16):
            nrm_st[pl.ds(b * B + k * 16, 16)] = 1.0 / jnp.maximum(
                cval2[b, pl.ds(k * 16, 16)], 1.0)
        return 0

    lax.fori_loop(0, NBW, mknrm, 0)
    pltpu.sync_copy(nrm_st, norm_hbm.at[pl.ds(base2, EDGES_PER_WORKER)])


NB = EDGES_PER_WORKER // B


@functools.partial(
    pl.kernel,
    out_type=jax.ShapeDtypeStruct((NC, N_ACC, C), jnp.float32),
    mesh=_mesh,
    scratch_types=[
        pltpu.VMEM_SHARED((N_ACC, C), jnp.float32),
        pltpu.VMEM((EDGES_PER_WORKER,), jnp.int32),
        pltpu.VMEM((EDGES_PER_WORKER,), jnp.float32),
        pltpu.VMEM((B,), jnp.int32),
        pltpu.VMEM((B,), jnp.int32),
        pltpu.VMEM((B, C), jnp.float32),
        pltpu.VMEM((B, C), jnp.float32),
        pltpu.SemaphoreType.DMA,
        pltpu.SemaphoreType.DMA,
        pltpu.SemaphoreType.DMA,
        pltpu.SemaphoreType.DMA,
    ],
)
def _sc_aggregate(hall_hbm, rid_hbm, dst_hbm, nrm_hbm, out_hbm,
                  acc_sh, rid_all, nrm_all,
                  dst_v0, dst_v1, rows0, rows1,
                  semg0, semg1, semd0, semd1):
    cid = lax.axis_index("c")
    sid = lax.axis_index("s")
    wid = sid * NC + cid
    base = wid * EDGES_PER_WORKER

    def zrow(i, _):
        for k in range(C // 16):
            rows0[i, pl.ds(k * 16, 16)] = jnp.zeros((16,), jnp.float32)
        return 0

    lax.fori_loop(0, B, zrow, 0)
    for i in range(ROWS_PER_TILE // B):
        pltpu.sync_copy(rows0, acc_sh.at[pl.ds(sid * ROWS_PER_TILE + i * B, B)])
    plsc.subcore_barrier()

    pltpu.sync_copy(rid_hbm.at[pl.ds(base, EDGES_PER_WORKER)], rid_all)
    pltpu.sync_copy(nrm_hbm.at[pl.ds(base, EDGES_PER_WORKER)], nrm_all)

    def fire(b, rows_buf, dst_buf, semg, semd):
        pltpu.async_copy(hall_hbm.at[rid_all.at[pl.ds(b * B, B)]], rows_buf, semg)
        pltpu.async_copy(dst_hbm.at[pl.ds(base + b * B, B)], dst_buf, semd)

    def consume(b, rows_buf, dst_buf, semg, semd):
        pltpu.make_async_copy(
            hall_hbm.at[rid_all.at[pl.ds(b * B, B)]], rows_buf, semg).wait()

        def scale_chunk(g, _):
            nv = nrm_all[pl.ds(b * B + g * 16, 16)]
            for j in range(16):
                row = g * 16 + j
                s = nv[j]
                for k in range(C // 16):
                    sl = pl.ds(k * 16, 16)
                    rows_buf[row, sl] = rows_buf[row, sl] * s
            return 0

        lax.fori_loop(0, B // 16, scale_chunk, 0)
        pltpu.make_async_copy(
            dst_hbm.at[pl.ds(base + b * B, B)], dst_buf, semd).wait()
        pltpu.sync_copy(rows_buf, acc_sh.at[dst_buf], add=True)

    fire(0, rows0, dst_v0, semg0, semd0)

    def pipe(g, _):
        b0 = g * 2
        fire(b0 + 1, rows1, dst_v1, semg1, semd1)
        consume(b0, rows0, dst_v0, semg0, semd0)
        fire(b0 + 2, rows0, dst_v0, semg0, semd0)
        consume(b0 + 1, rows1, dst_v1, semg1, semd1)
        return 0

    lax.fori_loop(0, (NB - 1) // 2, pipe, 0)
    consume(NB - 1, rows0, dst_v0, semg0, semd0)
    plsc.subcore_barrier()

    for i in range(ROWS_PER_TILE // ZROWS):
        r0 = sid * ROWS_PER_TILE + i * ZROWS
        pltpu.sync_copy(acc_sh.at[pl.ds(r0, ZROWS)], out_hbm.at[cid, pl.ds(r0, ZROWS)])


BN = 1000


def _rel_matmul_body(x_ref, w_ref, o_ref):
    o_ref[0] = jnp.dot(x_ref[...], w_ref[0], preferred_element_type=jnp.float32)


def _rel_matmul(x, W):
    return pl.pallas_call(
        _rel_matmul_body,
        grid=(R, N // BN),
        in_specs=[
            pl.BlockSpec((BN, C), lambda r, n: (n, 0)),
            pl.BlockSpec((1, C, C), lambda r, n: (r, 0, 0)),
        ],
        out_specs=pl.BlockSpec((1, BN, C), lambda r, n: (r, n, 0)),
        out_shape=jax.ShapeDtypeStruct((R, N, C), jnp.float32),
    )(x, W)


def _combine_relu_body(p0_ref, p1_ref, x_ref, w_ref, b_ref, o_ref):
    h = p0_ref[0] + p1_ref[0]
    h = h + jnp.dot(x_ref[...], w_ref[...], preferred_element_type=jnp.float32)
    h = h + b_ref[...]
    o_ref[...] = jnp.maximum(h, 0.0)


def _combine_norm_body(p0_ref, p1_ref, x_ref, w_ref, b_ref, o_ref):
    h = p0_ref[0] + p1_ref[0]
    h = h + jnp.dot(x_ref[...], w_ref[...], preferred_element_type=jnp.float32)
    h = h + b_ref[...]
    nrm = jnp.sqrt(jnp.sum(h * h, axis=-1, keepdims=True))
    o_ref[...] = h / jnp.maximum(nrm, 1e-12)


def _combine_call(parts, x, w_root, b, body):
    return pl.pallas_call(
        body,
        grid=(N // BN,),
        in_specs=[
            pl.BlockSpec((1, BN, C), lambda n: (0, n, 0)),
            pl.BlockSpec((1, BN, C), lambda n: (0, n, 0)),
            pl.BlockSpec((BN, C), lambda n: (n, 0)),
            pl.BlockSpec((C, C), lambda n: (0, 0)),
            pl.BlockSpec((1, C), lambda n: (0, 0)),
        ],
        out_specs=pl.BlockSpec((BN, C), lambda n: (n, 0)),
        out_shape=jax.ShapeDtypeStruct((N, C), jnp.float32),
    )(parts[0:1], parts[1:2], x, w_root, b[None, :])


def _layer(x, W, w_root, b, rid, dst, nrm, body):
    h_all = _rel_matmul(x, W).reshape(R * N, C)
    parts = _sc_aggregate(h_all, rid, dst, nrm)
    return _combine_call(parts, x, w_root, b, body)


@jax.jit
def kernel(x, edge_index, edge_type, W1, W_root1, b1, W2, W_root2, b2):
    src = edge_index[0]
    dst = edge_index[1]
    et = edge_type
    nrm, rid = _sc_structure(src, dst, et)
    h = _layer(x, W1, W_root1, b1, rid, dst, nrm, _combine_relu_body)
    out = _layer(h, W2, W_root2, b2, rid, dst, nrm, _combine_norm_body)
    return out


# speedup vs baseline: 32.1066x; 1.1092x over previous
import functools

import jax
import jax.numpy as jnp
from jax import lax
from jax.experimental import pallas as pl
from jax.experimental.pallas import tpu as pltpu
from jax.experimental.pallas import tpu_sc as plsc

N = 10000
E = 320000
R = 8
C = 128

NC = 2
NS = 16
NW = NC * NS

B = 80
EDGES_PER_TILE = E // NS
EDGES_PER_WORKER = E // NW
CNT_PAD = 81920
N_ACC = 10240
ROWS_PER_TILE = N_ACC // NS
ZROWS = 128

_mesh = plsc.VectorSubcoreMesh(core_axis_name="c", subcore_axis_name="s")


def _fill_const(ref, n16, value, dtype):
    v = jnp.full((16,), value, dtype=dtype)

    def body(i, _):
        ref[pl.ds(i * 16, 16)] = v
        return 0

    lax.fori_loop(0, n16, body, 0)


NBT = EDGES_PER_TILE // B
NBW = EDGES_PER_WORKER // B
W_IND = 5


@functools.partial(
    pl.kernel,
    out_type=[
        jax.ShapeDtypeStruct((E,), jnp.float32),
        jax.ShapeDtypeStruct((E,), jnp.int32),
    ],
    mesh=_mesh,
    scratch_types=[
        pltpu.VMEM_SHARED((CNT_PAD,), jnp.float32),
        pltpu.VMEM((CNT_PAD // NS,), jnp.float32),
        pltpu.VMEM((EDGES_PER_TILE,), jnp.int32),
        pltpu.VMEM((EDGES_PER_TILE,), jnp.int32),
        pltpu.VMEM((EDGES_PER_WORKER,), jnp.int32),
        pltpu.VMEM((NBT, B), jnp.int32),
        pltpu.VMEM((B,), jnp.float32),
        pltpu.VMEM((NBW, B), jnp.float32),
        pltpu.VMEM((EDGES_PER_WORKER,), jnp.float32),
        pltpu.VMEM((EDGES_PER_WORKER,), jnp.int32),
        pltpu.SemaphoreType.DMA,
    ],
)
def _sc_structure(src_hbm, dst_hbm, et_hbm, norm_hbm, rid_hbm,
                  cnt_sh, zbuf, d_all, t_all, s_all, comb2, ones_v,
                  cval2, nrm_st, rid_st, sem):
    cid = lax.axis_index("c")
    sid = lax.axis_index("s")
    wid = sid * NC + cid

    n16 = (CNT_PAD // NS) // 16
    _fill_const(zbuf, n16, 0.0, jnp.float32)
    pltpu.sync_copy(zbuf, cnt_sh.at[pl.ds(sid * (CNT_PAD // NS), CNT_PAD // NS)])
    _fill_const(ones_v, B // 16, 1.0, jnp.float32)
    plsc.subcore_barrier()

    base = sid * EDGES_PER_TILE
    pltpu.sync_copy(dst_hbm.at[pl.ds(base, EDGES_PER_TILE)], d_all)
    pltpu.sync_copy(et_hbm.at[pl.ds(base, EDGES_PER_TILE)], t_all)

    def mkcomb(b, _):
        for k in range(B // 16):
            sl = pl.ds(b * B + k * 16, 16)
            comb2[b, pl.ds(k * 16, 16)] = d_all[sl] * R + t_all[sl]
        return 0

    lax.fori_loop(0, NBT, mkcomb, 0)

    def cnt_wave(g, _):
        for j in range(W_IND):
            pltpu.async_copy(ones_v, cnt_sh.at[comb2.at[g * W_IND + j]], sem,
                             add=True)
        for j in range(W_IND):
            pltpu.make_async_copy(ones_v, cnt_sh.at[comb2.at[g * W_IND + j]],
                                  sem).wait()
        return 0

    lax.fori_loop(0, NBT // W_IND, cnt_wave, 0)
    plsc.subcore_barrier()

    base2 = wid * EDGES_PER_WORKER
    pltpu.sync_copy(dst_hbm.at[pl.ds(base2, EDGES_PER_WORKER)],
                    d_all.at[pl.ds(0, EDGES_PER_WORKER)])
    pltpu.sync_copy(et_hbm.at[pl.ds(base2, EDGES_PER_WORKER)],
                    t_all.at[pl.ds(0, EDGES_PER_WORKER)])
    pltpu.sync_copy(src_hbm.at[pl.ds(base2, EDGES_PER_WORKER)], s_all)

    def mkidx(b, _):
        for k in range(B // 16):
            sl = pl.ds(b * B + k * 16, 16)
            t = t_all[sl]
            comb2[b, pl.ds(k * 16, 16)] = d_all[sl] * R + t
            rid_st[sl] = s_all[sl] * R + t
        return 0

    lax.fori_loop(0, NBW, mkidx, 0)
    pltpu.sync_copy(rid_st, rid_hbm.at[pl.ds(base2, EDGES_PER_WORKER)])

    def gat_wave(g, _):
        for j in range(W_IND):
            b = g * W_IND + j
            pltpu.async_copy(cnt_sh.at[comb2.at[b]], cval2.at[b], sem)
        for j in range(W_IND):
            b = g * W_IND + j
            pltpu.make_async_copy(cnt_sh.at[comb2.at[b]], cval2.at[b],
                                  sem).wait()
        return 0

    lax.fori_loop(0, NBW // W_IND, gat_wave, 0)

    def mknrm(b, _):
        for k in range(B // 16):
            nrm_st[pl.ds(b * B + k * 16, 16)] = 1.0 / jnp.maximum(
                cval2[b, pl.ds(k * 16, 16)], 1.0)
        return 0

    lax.fori_loop(0, NBW, mknrm, 0)
    pltpu.sync_copy(nrm_st, norm_hbm.at[pl.ds(base2, EDGES_PER_WORKER)])


NB = EDGES_PER_WORKER // B


@functools.partial(
    pl.kernel,
    out_type=jax.ShapeDtypeStruct((NC, N_ACC, C), jnp.float32),
    mesh=_mesh,
    scratch_types=[
        pltpu.VMEM_SHARED((N_ACC, C), jnp.float32),
        pltpu.VMEM((EDGES_PER_WORKER,), jnp.int32),
        [pltpu.VMEM((B,), jnp.int32)] * 3,
        [pltpu.VMEM((B,), jnp.float32)] * 3,
        [pltpu.VMEM((B, C), jnp.float32)] * 3,
        [pltpu.SemaphoreType.DMA] * 3,
        [pltpu.SemaphoreType.DMA] * 3,
        [pltpu.SemaphoreType.DMA] * 3,
    ],
)
def _sc_aggregate(hall_hbm, rid_hbm, dst_hbm, nrm_hbm, out_hbm,
                  acc_sh, rid_all, dst_v, nrm_v, rows, semg, semm, sems):
    cid = lax.axis_index("c")
    sid = lax.axis_index("s")
    wid = sid * NC + cid
    base = wid * EDGES_PER_WORKER

    def zrow(i, _):
        for k in range(C // 16):
            rows[0][i, pl.ds(k * 16, 16)] = jnp.zeros((16,), jnp.float32)
        return 0

    lax.fori_loop(0, B, zrow, 0)
    for i in range(ROWS_PER_TILE // B):
        pltpu.sync_copy(rows[0], acc_sh.at[pl.ds(sid * ROWS_PER_TILE + i * B, B)])
    plsc.subcore_barrier()

    pltpu.sync_copy(rid_hbm.at[pl.ds(base, EDGES_PER_WORKER)], rid_all)

    def fire(b, j):
        pltpu.async_copy(hall_hbm.at[rid_all.at[pl.ds(b * B, B)]], rows[j],
                         semg[j])
        pltpu.async_copy(dst_hbm.at[pl.ds(base + b * B, B)], dst_v[j], semm[j])
        pltpu.async_copy(nrm_hbm.at[pl.ds(base + b * B, B)], nrm_v[j], semm[j])

    def consume(b, j):
        pltpu.make_async_copy(
            hall_hbm.at[rid_all.at[pl.ds(b * B, B)]], rows[j], semg[j]).wait()
        pltpu.make_async_copy(
            dst_hbm.at[pl.ds(base + b * B, B)], dst_v[j], semm[j]).wait()
        pltpu.make_async_copy(
            nrm_hbm.at[pl.ds(base + b * B, B)], nrm_v[j], semm[j]).wait()

        def scale_chunk(g, _):
            nv = nrm_v[j][pl.ds(g * 16, 16)]
            for jj in range(16):
                row = g * 16 + jj
                s = nv[jj]
                for k in range(C // 16):
                    sl = pl.ds(k * 16, 16)
                    rows[j][row, sl] = rows[j][row, sl] * s
            return 0

        lax.fori_loop(0, B // 16, scale_chunk, 0)
        pltpu.async_copy(rows[j], acc_sh.at[dst_v[j]], sems[j], add=True)

    def drain(j):
        pltpu.make_async_copy(rows[j], acc_sh.at[dst_v[j]], sems[j]).wait()

    fire(0, 0)
    fire(1, 1)
    fire(2, 2)
    consume(0, 0)

    def pipe(g, _):
        for db in range(3):
            b = 3 * g + 1 + db
            consume(b, (1 + db) % 3)
            drain(db % 3)
            fire(b + 2, db % 3)
        return 0

    lax.fori_loop(0, (NB - 5) // 3, pipe, 0)
    consume(121, 121 % 3)
    drain(120 % 3)
    fire(123, 123 % 3)
    consume(122, 122 % 3)
    drain(121 % 3)
    fire(124, 124 % 3)
    consume(123, 123 % 3)
    drain(122 % 3)
    consume(124, 124 % 3)
    drain(123 % 3)
    drain(124 % 3)
    plsc.subcore_barrier()

    for i in range(ROWS_PER_TILE // ZROWS):
        r0 = sid * ROWS_PER_TILE + i * ZROWS
        pltpu.sync_copy(acc_sh.at[pl.ds(r0, ZROWS)], out_hbm.at[cid, pl.ds(r0, ZROWS)])


BN = 1000


def _rel_matmul_body(x_ref, w_ref, o_ref):
    o_ref[...] = jnp.dot(x_ref[...], w_ref[...], preferred_element_type=jnp.float32)


def _rel_matmul(x, W2d):
    return pl.pallas_call(
        _rel_matmul_body,
        grid=(N // BN,),
        in_specs=[
            pl.BlockSpec((BN, C), lambda n: (n, 0)),
            pl.BlockSpec((C, R * C), lambda n: (0, 0)),
        ],
        out_specs=pl.BlockSpec((BN, R * C), lambda n: (n, 0)),
        out_shape=jax.ShapeDtypeStruct((N, R * C), jnp.float32),
    )(x, W2d)


def _combine_relu_body(p0_ref, p1_ref, x_ref, w_ref, b_ref, o_ref):
    h = p0_ref[0] + p1_ref[0]
    h = h + jnp.dot(x_ref[...], w_ref[...], preferred_element_type=jnp.float32)
    h = h + b_ref[...]
    o_ref[...] = jnp.maximum(h, 0.0)


def _combine_norm_body(p0_ref, p1_ref, x_ref, w_ref, b_ref, o_ref):
    h = p0_ref[0] + p1_ref[0]
    h = h + jnp.dot(x_ref[...], w_ref[...], preferred_element_type=jnp.float32)
    h = h + b_ref[...]
    nrm = jnp.sqrt(jnp.sum(h * h, axis=-1, keepdims=True))
    o_ref[...] = h / jnp.maximum(nrm, 1e-12)


def _combine_call(parts, x, w_root, b, body):
    return pl.pallas_call(
        body,
        grid=(N // BN,),
        in_specs=[
            pl.BlockSpec((1, BN, C), lambda n: (0, n, 0)),
            pl.BlockSpec((1, BN, C), lambda n: (0, n, 0)),
            pl.BlockSpec((BN, C), lambda n: (n, 0)),
            pl.BlockSpec((C, C), lambda n: (0, 0)),
            pl.BlockSpec((1, C), lambda n: (0, 0)),
        ],
        out_specs=pl.BlockSpec((BN, C), lambda n: (n, 0)),
        out_shape=jax.ShapeDtypeStruct((N, C), jnp.float32),
    )(parts[0:1], parts[1:2], x, w_root, b[None, :])


def _layer(x, W2d, w_root, b, rid, dst, nrm, body):
    h_all = _rel_matmul(x, W2d).reshape(N * R, C)
    parts = _sc_aggregate(h_all, rid, dst, nrm)
    return _combine_call(parts, x, w_root, b, body)


@jax.jit
def kernel(x, edge_index, edge_type, W1, W_root1, b1, W2, W_root2, b2):
    src = edge_index[0]
    dst = edge_index[1]
    et = edge_type
    W1_2d = W1.transpose(1, 0, 2).reshape(C, R * C)
    W2_2d = W2.transpose(1, 0, 2).reshape(C, R * C)
    nrm, rid = _sc_structure(src, dst, et)
    h = _layer(x, W1_2d, W_root1, b1, rid, dst, nrm, _combine_relu_body)
    out = _layer(h, W2_2d, W_root2, b2, rid, dst, nrm, _combine_norm_body)
    return out
